# baseline (speedup 1.0000x reference)
import jax
import jax.numpy as jnp
from jax import lax
from jax.experimental import pallas as pl
from jax.experimental.pallas import tpu as pltpu

_B, _D, _C, _NGT = 32, 8732, 81, 16
_THRESHOLD = 0.5
_RATIO = 3.0
_ALPHA = 1.0
_BS_ITERS = 7


def _loss_kernel(pbt_ref, plabt_ref, bx_ref, lab_ref, dbt_ref, out_ref):
    i = pl.program_id(0)
    f32 = jnp.float32

    b = bx_ref[0]
    labels_g = lab_ref[0]
    b0 = b[:, 0:1]
    b1 = b[:, 1:2]
    b2 = b[:, 2:3]
    b3 = b[:, 3:4]

    dbt = dbt_ref[...]
    dbx = dbt[0:1, :]
    dby = dbt[1:2, :]
    dbw = dbt[2:3, :]
    dbh = dbt[3:4, :]
    ux = dbx - dbw * 0.5
    uy = dby - dbh * 0.5

    lbx = jnp.maximum(b0, ux)
    lby = jnp.maximum(b1, uy)
    ubx = jnp.minimum(b2, dbw)
    uby = jnp.minimum(b3, dbh)
    iw = jnp.maximum(ubx - lbx, 0.0)
    ih = jnp.maximum(uby - lby, 0.0)
    inter = iw * ih
    ar1 = (b2 - b0) * (b3 - b1)
    ar2 = (dbw - ux) * (dbh - uy)
    iou = inter / (ar1 + ar2 - inter)

    g_iota = lax.broadcasted_iota(jnp.int32, (_NGT, _D), 0)
    d_iota = lax.broadcasted_iota(jnp.int32, (_NGT, _D), 1)
    db_val = jnp.max(iou, axis=0, keepdims=True)
    db_box = jnp.min(jnp.where(iou == db_val, g_iota, _NGT), axis=0, keepdims=True)
    row_max = jnp.max(iou, axis=1, keepdims=True)
    box_db = jnp.min(jnp.where(iou == row_max, d_iota, _D), axis=1, keepdims=True)

    d_row = lax.broadcasted_iota(jnp.int32, (1, _D), 1)
    match = d_row == box_db
    lastg = jnp.max(jnp.where(match, g_iota, -1), axis=0, keepdims=True)
    forced = lastg >= 0
    db_box = jnp.where(forced, lastg, db_box)
    db_val = jnp.where(forced, _THRESHOLD, db_val)

    g_col = lax.broadcasted_iota(jnp.int32, (_NGT, 1), 0)
    onehot_b = db_box == g_col
    lab = jnp.max(jnp.where(onehot_b, jnp.broadcast_to(labels_g, (_NGT, _D)), 0),
                  axis=0, keepdims=True)
    lab = jnp.where(db_val < _THRESHOLD, 0, lab)
    onehot = onehot_b.astype(f32)
    s0 = jnp.sum(onehot * b0, axis=0, keepdims=True)
    s1 = jnp.sum(onehot * b1, axis=0, keepdims=True)
    s2 = jnp.sum(onehot * b2, axis=0, keepdims=True)
    s3 = jnp.sum(onehot * b3, axis=0, keepdims=True)

    gcx = (s0 + s2) * 0.5
    gcy = (s1 + s3) * 0.5
    t0 = (gcx - dbx) / dbw
    t1 = (gcy - dby) / dbh
    t2 = jnp.log(s2 / dbw)
    t3 = jnp.log(s3 / dbh)

    mask = (lab != 0).astype(f32)
    n_pos = jnp.sum(mask)

    pbt = pbt_ref[0]
    abs_sum = (jnp.sum(jnp.abs(pbt[0:1, :] - t0) * mask)
               + jnp.sum(jnp.abs(pbt[1:2, :] - t1) * mask)
               + jnp.sum(jnp.abs(pbt[2:3, :] - t2) * mask)
               + jnp.sum(jnp.abs(pbt[3:4, :] - t3) * mask))

    x = plabt_ref[0]
    m = jnp.max(x, axis=0, keepdims=True)
    se = jnp.sum(jnp.exp(x - m), axis=0, keepdims=True)
    lse = m + jnp.log(se)
    c_col = lax.broadcasted_iota(jnp.int32, (_C, _D), 0)
    picked = jnp.sum(jnp.where(c_col == lab, x, 0.0), axis=0, keepdims=True)
    closs = lse - picked
    pos_sum = jnp.sum(closs * mask)
    neg = closs * (1.0 - mask)

    K = jnp.minimum(_RATIO * n_pos, float(_D))
    hi0 = jnp.max(neg)
    steps = ((lax.broadcasted_iota(jnp.int32, (32, 1), 0).astype(f32) + 1.0)
             * (1.0 / 33.0))

    def bs_body(_, carry):
        lo, hi = carry
        ts = lo + (hi - lo) * steps
        cnts = jnp.sum((neg > ts).astype(f32), axis=1, keepdims=True)
        above = cnts > K
        new_lo = jnp.max(jnp.where(above, ts, lo))
        new_hi = jnp.min(jnp.where(above, hi, ts))
        return new_lo, new_hi

    lo, hi = lax.fori_loop(0, _BS_ITERS, bs_body,
                           (jnp.float32(0.0), hi0))
    t = hi
    gtmask = neg > t
    cnt_gt = jnp.sum(gtmask.astype(f32))
    sum_gt = jnp.sum(jnp.where(gtmask, neg, 0.0))
    neg_top = sum_gt + jnp.maximum(K - cnt_gt, 0.0) * t

    lane = lax.broadcasted_iota(jnp.int32, (1, 128), 1)
    contrib = (jnp.where(lane == 0, abs_sum, 0.0)
               + jnp.where(lane == 1, n_pos, 0.0)
               + jnp.where(lane == 2, pos_sum, 0.0)
               + jnp.where(lane == 3, neg_top, 0.0))

    @pl.when(i == 0)
    def _():
        out_ref[...] = contrib

    @pl.when(i > 0)
    def _():
        out_ref[...] = out_ref[...] + contrib


_CHUNKS = 4
_CB = _B // _CHUNKS


def kernel(predicted_boxes, predicted_labels, boxes, labels, default_boxes):
    labels3 = labels[:, :, None]
    dbt = default_boxes.T

    call = pl.pallas_call(
        _loss_kernel,
        grid=(_CB,),
        in_specs=[
            pl.BlockSpec((1, 4, _D), lambda i: (i, 0, 0)),
            pl.BlockSpec((1, _C, _D), lambda i: (i, 0, 0)),
            pl.BlockSpec((1, _NGT, 4), lambda i: (i, 0, 0)),
            pl.BlockSpec((1, _NGT, 1), lambda i: (i, 0, 0)),
            pl.BlockSpec((4, _D), lambda i: (0, 0)),
        ],
        out_specs=pl.BlockSpec((1, 128), lambda i: (0, 0)),
        out_shape=jax.ShapeDtypeStruct((1, 128), jnp.float32),
        compiler_params=pltpu.CompilerParams(
            dimension_semantics=("arbitrary",),
        ),
    )

    out = None
    for k in range(_CHUNKS):
        sl = slice(k * _CB, (k + 1) * _CB)
        pbt_k = jnp.swapaxes(predicted_boxes[sl], 1, 2)
        plabt_k = jnp.swapaxes(predicted_labels[sl], 1, 2)
        o = call(pbt_k, plabt_k, boxes[sl], labels3[sl], dbt)
        out = o if out is None else out + o

    r = out[0]
    abs_sum, n_pos_sum, pos_sum, neg_sum = r[0], r[1], r[2], r[3]
    box_loss = abs_sum / (n_pos_sum * 4.0)
    return (neg_sum + pos_sum) / n_pos_sum + _ALPHA * box_loss


# speedup vs baseline: 1.1679x; 1.1679x over previous
import jax
import jax.numpy as jnp
from jax import lax
from jax.experimental import pallas as pl
from jax.experimental.pallas import tpu as pltpu

_B, _D, _C, _NGT = 32, 8732, 81, 16
_THRESHOLD = 0.5
_RATIO = 3.0
_ALPHA = 1.0
_BS_ITERS = 7
_IPS = 2


def _one_image(pbt, x, b, labels_g, dbt):
    f32 = jnp.float32

    b0 = b[:, 0:1]
    b1 = b[:, 1:2]
    b2 = b[:, 2:3]
    b3 = b[:, 3:4]

    dbx = dbt[0:1, :]
    dby = dbt[1:2, :]
    dbw = dbt[2:3, :]
    dbh = dbt[3:4, :]
    ux = dbx - dbw * 0.5
    uy = dby - dbh * 0.5

    lbx = jnp.maximum(b0, ux)
    lby = jnp.maximum(b1, uy)
    ubx = jnp.minimum(b2, dbw)
    uby = jnp.minimum(b3, dbh)
    iw = jnp.maximum(ubx - lbx, 0.0)
    ih = jnp.maximum(uby - lby, 0.0)
    inter = iw * ih
    ar1 = (b2 - b0) * (b3 - b1)
    ar2 = (dbw - ux) * (dbh - uy)
    iou = inter / (ar1 + ar2 - inter)

    g_iota = lax.broadcasted_iota(jnp.int32, (_NGT, _D), 0)
    d_iota = lax.broadcasted_iota(jnp.int32, (_NGT, _D), 1)
    db_val = jnp.max(iou, axis=0, keepdims=True)
    db_box = jnp.min(jnp.where(iou == db_val, g_iota, _NGT), axis=0, keepdims=True)
    row_max = jnp.max(iou, axis=1, keepdims=True)
    box_db = jnp.min(jnp.where(iou == row_max, d_iota, _D), axis=1, keepdims=True)

    d_row = lax.broadcasted_iota(jnp.int32, (1, _D), 1)
    match = d_row == box_db
    lastg = jnp.max(jnp.where(match, g_iota, -1), axis=0, keepdims=True)
    forced = lastg >= 0
    db_box = jnp.where(forced, lastg, db_box)
    db_val = jnp.where(forced, _THRESHOLD, db_val)

    g_col = lax.broadcasted_iota(jnp.int32, (_NGT, 1), 0)
    onehot_b = db_box == g_col
    lab = jnp.max(jnp.where(onehot_b, jnp.broadcast_to(labels_g, (_NGT, _D)), 0),
                  axis=0, keepdims=True)
    lab = jnp.where(db_val < _THRESHOLD, 0, lab)
    onehot = onehot_b.astype(f32)
    s0 = jnp.sum(onehot * b0, axis=0, keepdims=True)
    s1 = jnp.sum(onehot * b1, axis=0, keepdims=True)
    s2 = jnp.sum(onehot * b2, axis=0, keepdims=True)
    s3 = jnp.sum(onehot * b3, axis=0, keepdims=True)

    gcx = (s0 + s2) * 0.5
    gcy = (s1 + s3) * 0.5
    t0 = (gcx - dbx) / dbw
    t1 = (gcy - dby) / dbh
    t2 = jnp.log(s2 / dbw)
    t3 = jnp.log(s3 / dbh)

    mask = (lab != 0).astype(f32)
    n_pos = jnp.sum(mask)

    abs_sum = (jnp.sum(jnp.abs(pbt[0:1, :] - t0) * mask)
               + jnp.sum(jnp.abs(pbt[1:2, :] - t1) * mask)
               + jnp.sum(jnp.abs(pbt[2:3, :] - t2) * mask)
               + jnp.sum(jnp.abs(pbt[3:4, :] - t3) * mask))

    m = jnp.max(x, axis=0, keepdims=True)
    se = jnp.sum(jnp.exp(x - m), axis=0, keepdims=True)
    lse = m + jnp.log(se)
    c_col = lax.broadcasted_iota(jnp.int32, (_C, _D), 0)
    picked = jnp.sum(jnp.where(c_col == lab, x, 0.0), axis=0, keepdims=True)
    closs = lse - picked
    pos_sum = jnp.sum(closs * mask)
    neg = closs * (1.0 - mask)

    K = jnp.minimum(_RATIO * n_pos, float(_D))
    hi0 = jnp.max(neg)
    steps = ((lax.broadcasted_iota(jnp.int32, (32, 1), 0).astype(f32) + 1.0)
             * (1.0 / 33.0))

    def bs_body(_, carry):
        lo, hi = carry
        ts = lo + (hi - lo) * steps
        cnts = jnp.sum((neg > ts).astype(f32), axis=1, keepdims=True)
        above = cnts > K
        new_lo = jnp.max(jnp.where(above, ts, lo))
        new_hi = jnp.min(jnp.where(above, hi, ts))
        return new_lo, new_hi

    lo, hi = lax.fori_loop(0, _BS_ITERS, bs_body,
                           (jnp.float32(0.0), hi0))
    t = hi
    gtmask = neg > t
    cnt_gt = jnp.sum(gtmask.astype(f32))
    sum_gt = jnp.sum(jnp.where(gtmask, neg, 0.0))
    neg_top = sum_gt + jnp.maximum(K - cnt_gt, 0.0) * t

    return abs_sum, n_pos, pos_sum, neg_top


def _loss_kernel(pbt_ref, plabt_ref, bx_ref, lab_ref, dbt_ref, out_ref):
    i = pl.program_id(0)
    dbt = dbt_ref[...]

    parts = [_one_image(pbt_ref[j], plabt_ref[j], bx_ref[j], lab_ref[j], dbt)
             for j in range(_IPS)]
    abs_sum = sum(p[0] for p in parts)
    n_pos = sum(p[1] for p in parts)
    pos_sum = sum(p[2] for p in parts)
    neg_top = sum(p[3] for p in parts)

    lane = lax.broadcasted_iota(jnp.int32, (1, 128), 1)
    contrib = (jnp.where(lane == 0, abs_sum, 0.0)
               + jnp.where(lane == 1, n_pos, 0.0)
               + jnp.where(lane == 2, pos_sum, 0.0)
               + jnp.where(lane == 3, neg_top, 0.0))

    @pl.when(i == 0)
    def _():
        out_ref[...] = contrib

    @pl.when(i > 0)
    def _():
        out_ref[...] = out_ref[...] + contrib


def kernel(predicted_boxes, predicted_labels, boxes, labels, default_boxes):
    pbt = jnp.swapaxes(predicted_boxes, 1, 2)
    plabt = jnp.swapaxes(predicted_labels, 1, 2)
    labels3 = labels[:, :, None]
    dbt = default_boxes.T

    out = pl.pallas_call(
        _loss_kernel,
        grid=(_B // _IPS,),
        in_specs=[
            pl.BlockSpec((_IPS, 4, _D), lambda i: (i, 0, 0)),
            pl.BlockSpec((_IPS, _C, _D), lambda i: (i, 0, 0)),
            pl.BlockSpec((_IPS, _NGT, 4), lambda i: (i, 0, 0)),
            pl.BlockSpec((_IPS, _NGT, 1), lambda i: (i, 0, 0)),
            pl.BlockSpec((4, _D), lambda i: (0, 0)),
        ],
        out_specs=pl.BlockSpec((1, 128), lambda i: (0, 0)),
        out_shape=jax.ShapeDtypeStruct((1, 128), jnp.float32),
        compiler_params=pltpu.CompilerParams(
            dimension_semantics=("arbitrary",),
        ),
    )(pbt, plabt, boxes, labels3, dbt)

    r = out[0]
    abs_sum, n_pos_sum, pos_sum, neg_sum = r[0], r[1], r[2], r[3]
    box_loss = abs_sum / (n_pos_sum * 4.0)
    return (neg_sum + pos_sum) / n_pos_sum + _ALPHA * box_loss
